# baseline (speedup 1.0000x reference)
import functools

import jax
import jax.numpy as jnp
from jax import lax
from jax.experimental import pallas as pl
from jax.experimental.pallas import tpu as pltpu
from jax.experimental.pallas import tpu_sc as plsc

B = 128
T = 4096
F = 16
SEASON_LEN = 24
C_OUT = 19
LANES = 16
NUM_CORES = 2
NUM_SUBCORES = 16
NUM_WORKERS = NUM_CORES * NUM_SUBCORES
BATCH_PER_WORKER = B // NUM_WORKERS
NUM_CHUNKS = -(-(T - SEASON_LEN) // LANES)
NSLAB = 4
TS = T // NSLAB


def _pow_iota(base, iota):
    r = jnp.ones((LANES,), jnp.float32)
    b = base
    for bit in range(4):
        m = ((iota >> bit) & 1) == 1
        r = jnp.where(m, r * b, r)
        b = b * b
    return r


def _hw_body(in_hbm, a_hbm, g_hbm, out_hbm, out_v, sin_v, a_v, g_v):
    cid = lax.axis_index("c")
    sid = lax.axis_index("s")
    wid = sid * NUM_CORES + cid

    pltpu.sync_copy(a_hbm, a_v)
    pltpu.sync_copy(g_hbm, g_v)
    av = a_v[...]
    gv = g_v[...]
    oma = 1.0 - av
    omg = 1.0 - gv
    iota = lax.iota(jnp.int32, LANES)
    pw = _pow_iota(oma, iota)
    ipw = _pow_iota(1.0 / oma, iota)

    zeros = jnp.zeros((LANES,), jnp.float32)
    m8 = iota < (SEASON_LEN - LANES)

    for bi in range(BATCH_PER_WORKER):
        b = wid * BATCH_PER_WORKER + bi

        for sl in range(NSLAB):
            pltpu.sync_copy(
                in_hbm.at[pl.ds(b * T * F + sl * TS * F, TS * F)], sin_v)

            @plsc.parallel_loop(0, TS, 1, unroll=8)
            def restripe(t):
                row = sin_v[pl.ds(t * F, LANES)]
                dst = ((sl * TS + t) * C_OUT + 1) + iota
                plsc.store_scatter(out_v, [dst], row)

        f0 = iota * C_OUT
        f1 = (iota + LANES) * C_OUT
        p0 = plsc.load_gather(out_v, [f0 + 1])
        p1 = plsc.load_gather(out_v, [f1 + 1])
        init = (jnp.sum(p0) + jnp.sum(jnp.where(m8, p1, 0.0))) * (
            1.0 / SEASON_LEN)
        init_v = lax.broadcast(init, (LANES,))
        plsc.store_scatter(out_v, [f0], p0)
        plsc.store_scatter(out_v, [f0 + (C_OUT - 2)], init_v)
        plsc.store_scatter(out_v, [f0 + (C_OUT - 1)], zeros)
        plsc.store_scatter(out_v, [f1], p1, mask=m8)
        plsc.store_scatter(out_v, [f1 + (C_OUT - 2)], init_v, mask=m8)
        plsc.store_scatter(out_v, [f1 + (C_OUT - 1)], zeros, mask=m8)

        def chunk(i, lprev):
            t0 = SEASON_LEN + LANES * i
            rows_raw = t0 + iota
            valid = rows_raw < T
            rows = jnp.minimum(rows_raw, T - 1)
            fr = rows * C_OUT
            p = plsc.load_gather(out_v, [fr + 1])
            slag = plsc.load_gather(
                out_v, [fr - (SEASON_LEN * C_OUT - (C_OUT - 1))])
            w = av * (p - slag) * ipw
            cs = plsc.cumsum(w)
            l = pw * (oma * lprev + cs)
            s = gv * (p - l) + omg * slag
            y = p - s
            plsc.store_scatter(out_v, [fr], y, mask=valid)
            plsc.store_scatter(out_v, [fr + (C_OUT - 2)], l, mask=valid)
            plsc.store_scatter(out_v, [fr + (C_OUT - 1)], s, mask=valid)
            return jnp.sum(jnp.where(iota == LANES - 1, l, 0.0))

        lax.fori_loop(0, NUM_CHUNKS, chunk, init)

        pltpu.sync_copy(out_v, out_hbm.at[pl.ds(b * T * C_OUT, T * C_OUT)])


def kernel(inputs, alpha, gamma):
    mesh = plsc.VectorSubcoreMesh(
        core_axis_name="c", subcore_axis_name="s",
        num_cores=NUM_CORES, num_subcores=NUM_SUBCORES)
    hw = functools.partial(
        pl.kernel,
        out_type=jax.ShapeDtypeStruct((B * T * C_OUT,), jnp.float32),
        mesh=mesh,
        scratch_types=[
            pltpu.VMEM((T * C_OUT,), jnp.float32),
            pltpu.VMEM((TS * F,), jnp.float32),
            pltpu.VMEM((LANES,), jnp.float32),
            pltpu.VMEM((LANES,), jnp.float32),
        ],
        compiler_params=pltpu.CompilerParams(
            needs_layout_passes=False, use_tc_tiling_on_sc=False),
    )(_hw_body)
    a16 = jnp.broadcast_to(alpha.astype(jnp.float32), (LANES,))
    g16 = jnp.broadcast_to(gamma.astype(jnp.float32), (LANES,))
    out2 = hw(inputs.reshape(B * T * F), a16, g16)
    return out2.reshape(B, T, C_OUT)


# speedup vs baseline: 4.5776x; 4.5776x over previous
import functools

import jax
import jax.numpy as jnp
from jax import lax
from jax.experimental import pallas as pl
from jax.experimental.pallas import tpu as pltpu
from jax.experimental.pallas import tpu_sc as plsc

B = 128
T = 4096
F = 16
SEASON_LEN = 24
C_OUT = 19
LANES = 16
NUM_CORES = 2
NUM_SUBCORES = 16
NUM_WORKERS = NUM_CORES * NUM_SUBCORES
BATCH_PER_WORKER = B // NUM_WORKERS
NUM_CHUNKS = -(-(T - SEASON_LEN) // LANES)
T_ALLOC = 4112


def _pow_iota(base, iota):
    r = jnp.ones((LANES,), jnp.float32)
    b = base
    for bit in range(4):
        m = ((iota >> bit) & 1) == 1
        r = jnp.where(m, r * b, r)
        b = b * b
    return r


def _hw_body(in_hbm, a_hbm, g_hbm, out_hbm, in_v, y_v, l_v, s_v, a_v, g_v):
    cid = lax.axis_index("c")
    sid = lax.axis_index("s")
    wid = sid * NUM_CORES + cid

    pltpu.sync_copy(a_hbm, a_v)
    pltpu.sync_copy(g_hbm, g_v)
    av = a_v[...]
    gv = g_v[...]
    oma = 1.0 - av
    omg = 1.0 - gv
    iota = lax.iota(jnp.int32, LANES)
    pw = _pow_iota(oma, iota)
    ipw = _pow_iota(1.0 / oma, iota)
    m8 = iota < (SEASON_LEN - LANES)

    for bi in range(BATCH_PER_WORKER):
        b = wid * BATCH_PER_WORKER + bi
        pltpu.sync_copy(in_hbm.at[b], in_v)
        for c in range(F):
            pltpu.sync_copy(in_v.at[c], out_hbm.at[c + 1, b])

        p0 = in_v[0, pl.ds(0, LANES)]
        p1 = in_v[0, pl.ds(LANES, LANES)]
        init = (jnp.sum(p0) + jnp.sum(jnp.where(m8, p1, 0.0))) * (
            1.0 / SEASON_LEN)
        init_v = lax.broadcast(init, (LANES,))
        zeros = jnp.zeros((LANES,), jnp.float32)
        y_v[pl.ds(0, LANES)] = p0
        y_v[pl.ds(LANES, LANES)] = p1
        l_v[pl.ds(0, LANES)] = init_v
        l_v[pl.ds(LANES, LANES)] = init_v
        s_v[pl.ds(0, LANES)] = zeros
        s_v[pl.ds(LANES, LANES)] = zeros

        def chunk(i, lprev):
            t0 = SEASON_LEN + LANES * i
            p = in_v[0, pl.ds(t0, LANES)]
            slag = s_v[pl.ds(t0 - SEASON_LEN, LANES)]
            w = av * (p - slag) * ipw
            cs = plsc.cumsum(w)
            l = pw * (oma * lprev + cs)
            s = gv * (p - l) + omg * slag
            y = p - s
            y_v[pl.ds(t0, LANES)] = y
            l_v[pl.ds(t0, LANES)] = l
            s_v[pl.ds(t0, LANES)] = s
            return jnp.sum(jnp.where(iota == LANES - 1, l, 0.0))

        lax.fori_loop(0, NUM_CHUNKS, chunk, init)

        pltpu.sync_copy(y_v.at[pl.ds(0, T)], out_hbm.at[0, b])
        pltpu.sync_copy(l_v.at[pl.ds(0, T)], out_hbm.at[C_OUT - 2, b])
        pltpu.sync_copy(s_v.at[pl.ds(0, T)], out_hbm.at[C_OUT - 1, b])


def kernel(inputs, alpha, gamma):
    mesh = plsc.VectorSubcoreMesh(
        core_axis_name="c", subcore_axis_name="s",
        num_cores=NUM_CORES, num_subcores=NUM_SUBCORES)
    hw = functools.partial(
        pl.kernel,
        out_type=jax.ShapeDtypeStruct((C_OUT, B, T), jnp.float32),
        mesh=mesh,
        scratch_types=[
            pltpu.VMEM((F, T), jnp.float32),
            pltpu.VMEM((T_ALLOC,), jnp.float32),
            pltpu.VMEM((T_ALLOC,), jnp.float32),
            pltpu.VMEM((T_ALLOC,), jnp.float32),
            pltpu.VMEM((LANES,), jnp.float32),
            pltpu.VMEM((LANES,), jnp.float32),
        ],
        compiler_params=pltpu.CompilerParams(
            needs_layout_passes=False, use_tc_tiling_on_sc=False),
    )(_hw_body)
    a16 = jnp.broadcast_to(alpha.astype(jnp.float32), (LANES,))
    g16 = jnp.broadcast_to(gamma.astype(jnp.float32), (LANES,))
    out_t = hw(jnp.transpose(inputs, (0, 2, 1)), a16, g16)
    return jnp.transpose(out_t, (1, 2, 0))


# speedup vs baseline: 8.5019x; 1.8573x over previous
import functools

import jax
import jax.numpy as jnp
from jax import lax
from jax.experimental import pallas as pl
from jax.experimental.pallas import tpu as pltpu
from jax.experimental.pallas import tpu_sc as plsc

B = 128
T = 4096
F = 16
SEASON_LEN = 24
C_OUT = 19
LANES = 16
NUM_CORES = 2
NUM_SUBCORES = 16
NUM_WORKERS = NUM_CORES * NUM_SUBCORES
BATCH_PER_WORKER = B // NUM_WORKERS
NUM_CHUNKS = T // LANES
TGRP = T // 128


def _pow_e(base, e):
    r = jnp.ones((LANES,), jnp.float32)
    b = base
    for bit in range(5):
        m = ((e >> bit) & 1) == 1
        r = jnp.where(m, r * b, r)
        b = b * b
    return r


def _hw_body(in_hbm, a_hbm, g_hbm, out_hbm, in_v, y5_v, l5_v, s5_v, s_v,
             a_v, g_v):
    cid = lax.axis_index("c")
    sid = lax.axis_index("s")
    wid = sid * NUM_CORES + cid

    pltpu.sync_copy(a_hbm, a_v)
    pltpu.sync_copy(g_hbm, g_v)
    av = a_v[...]
    gv = g_v[...]
    oma = 1.0 - av
    omg = 1.0 - gv
    iota = lax.iota(jnp.int32, LANES)
    pw = _pow_e(oma, iota + 1)
    ipw = _pow_e(1.0 / oma, iota + 1)
    e_mix = jnp.maximum(0, iota - (SEASON_LEN - LANES - 1))
    pw_mix = _pow_e(oma, e_mix)
    ipw_mix = _pow_e(1.0 / oma, e_mix)
    act = iota >= (SEASON_LEN - LANES)
    m8 = iota < (SEASON_LEN - LANES)

    for bi_ in range(BATCH_PER_WORKER):
        b = wid * BATCH_PER_WORKER + bi_
        bg = b // 8
        bi = b % 8
        pltpu.sync_copy(in_hbm.at[b], in_v)
        for c in range(F):
            pltpu.sync_copy(
                in_v.at[c // 8, :, c % 8:c % 8 + 1, :],
                out_hbm.at[c + 1, bg, :, pl.ds(bi, 1), :])

        p0 = in_v[0, 0, 0, pl.ds(0, LANES)]
        p1 = in_v[0, 0, 0, pl.ds(LANES, LANES)]
        init = (jnp.sum(p0) + jnp.sum(jnp.where(m8, p1, 0.0))) * (
            1.0 / SEASON_LEN)
        init_v = lax.broadcast(init, (LANES,))
        zeros = jnp.zeros((LANES,), jnp.float32)
        y5_v[0, 0, pl.ds(0, LANES)] = p0
        l5_v[0, 0, pl.ds(0, LANES)] = init_v
        s5_v[0, 0, pl.ds(0, LANES)] = zeros
        s_v[pl.ds(0, LANES)] = zeros

        z1 = jnp.where(act, p1, 0.0)
        cs1 = plsc.cumsum(av * z1 * ipw_mix)
        l1 = pw_mix * (init + cs1)
        s1 = jnp.where(act, gv * (p1 - l1), 0.0)
        y1 = p1 - s1
        y5_v[0, 0, pl.ds(LANES, LANES)] = y1
        l5_v[0, 0, pl.ds(LANES, LANES)] = l1
        s5_v[0, 0, pl.ds(LANES, LANES)] = s1
        s_v[pl.ds(LANES, LANES)] = s1

        def chunk(i, lprev):
            t0 = LANES * i
            tg = t0 >> 7
            ti = t0 & 127
            p = in_v[0, tg, 0, pl.ds(ti, LANES)]
            slag = s_v[pl.ds(t0 - SEASON_LEN, LANES)]
            w = av * (p - slag) * ipw
            cs = plsc.cumsum(w)
            l = pw * (lprev + cs)
            s = gv * (p - l) + omg * slag
            y = p - s
            y5_v[tg, 0, pl.ds(ti, LANES)] = y
            l5_v[tg, 0, pl.ds(ti, LANES)] = l
            s5_v[tg, 0, pl.ds(ti, LANES)] = s
            s_v[pl.ds(t0, LANES)] = s
            return jnp.sum(jnp.where(iota == LANES - 1, l, 0.0))

        lax.fori_loop(2, NUM_CHUNKS, chunk,
                      jnp.sum(jnp.where(iota == LANES - 1, l1, 0.0)))

        pltpu.sync_copy(y5_v, out_hbm.at[0, bg, :, pl.ds(bi, 1), :])
        pltpu.sync_copy(l5_v, out_hbm.at[C_OUT - 2, bg, :, pl.ds(bi, 1), :])
        pltpu.sync_copy(s5_v, out_hbm.at[C_OUT - 1, bg, :, pl.ds(bi, 1), :])


def kernel(inputs, alpha, gamma):
    mesh = plsc.VectorSubcoreMesh(
        core_axis_name="c", subcore_axis_name="s",
        num_cores=NUM_CORES, num_subcores=NUM_SUBCORES)
    hw = functools.partial(
        pl.kernel,
        out_type=jax.ShapeDtypeStruct((C_OUT, B // 8, TGRP, 8, 128),
                                      jnp.float32),
        mesh=mesh,
        scratch_types=[
            pltpu.VMEM((F // 8, TGRP, 8, 128), jnp.float32),
            pltpu.VMEM((TGRP, 1, 128), jnp.float32),
            pltpu.VMEM((TGRP, 1, 128), jnp.float32),
            pltpu.VMEM((TGRP, 1, 128), jnp.float32),
            pltpu.VMEM((T,), jnp.float32),
            pltpu.VMEM((LANES,), jnp.float32),
            pltpu.VMEM((LANES,), jnp.float32),
        ],
        compiler_params=pltpu.CompilerParams(
            needs_layout_passes=False, use_tc_tiling_on_sc=False),
    )(_hw_body)
    a16 = jnp.broadcast_to(alpha.astype(jnp.float32), (LANES,))
    g16 = jnp.broadcast_to(gamma.astype(jnp.float32), (LANES,))
    in5 = jnp.transpose(
        jnp.reshape(jnp.transpose(inputs, (0, 2, 1)),
                    (B, F // 8, 8, TGRP, 128)),
        (0, 1, 3, 2, 4))
    out5 = hw(in5, a16, g16)
    out = jnp.reshape(jnp.transpose(out5, (1, 3, 2, 4, 0)),
                      (B, T, C_OUT))
    return out
